# baseline (speedup 1.0000x reference)
import numpy as np
import jax
import jax.numpy as jnp
from jax.experimental import pallas as pl
from jax.experimental.pallas import tpu as pltpu

_B, _C, _T = 128, 128, 2048
_NL = 527


def _mix_plan():
    rs = np.random.RandomState(seed=1234)
    rs.uniform()
    rs.uniform()
    lam = rs.beta(0.3, 0.3)
    perm = rs.permutation(_B)
    order, is_head = [], []
    visited = np.zeros(_B, bool)
    for s in range(_B):
        if visited[s]:
            continue
        e = s
        first = True
        while not visited[e]:
            visited[e] = True
            order.append(int(e))
            is_head.append(1 if first else 0)
            first = False
            e = int(perm[e])
    fsrc = np.asarray(order + [0], np.int32)
    head = np.asarray(is_head + [1], np.int32)
    dst = np.asarray([order[0]] + order, np.int32)
    m = np.zeros((_B, _B), np.float32)
    m[np.arange(_B), np.arange(_B)] += np.float32(1.0 - lam)
    m[np.arange(_B), perm] += np.float32(lam)
    return float(lam), m, fsrc, head, dst


_LAM, _MIX, _FSRC, _HEAD, _DST = _mix_plan()
_G = len(_FSRC)


def _body(fsrc_ref, head_ref, dst_ref, x_hbm, m_ref, y_ref, ox_ref, oy_ref,
          ring, headbuf, sems):
    g = pl.program_id(0)
    slot = jax.lax.rem(g, 3)
    nxt = jax.lax.rem(g + 1, 3)
    prv = jax.lax.rem(g + 2, 3)
    nxt_src = fsrc_ref[jnp.minimum(g + 1, _G - 1)]
    at_head = head_ref[g]

    @pl.when(g == 0)
    def _():
        pltpu.make_async_copy(x_hbm.at[fsrc_ref[0]], ring.at[0], sems.at[0]).start()

    @pl.when(g < _B - 1)
    def _():
        pltpu.make_async_copy(x_hbm.at[nxt_src], ring.at[nxt], sems.at[nxt]).start()

    @pl.when(g < _B)
    def _():
        pltpu.make_async_copy(x_hbm.at[fsrc_ref[g]], ring.at[slot], sems.at[slot]).wait()

    @pl.when(jnp.logical_and(g > 0, at_head == 0))
    def _():
        ox_ref[0] = (1.0 - _LAM) * ring[prv] + _LAM * ring[slot]

    @pl.when(jnp.logical_and(g > 0, at_head == 1))
    def _():
        ox_ref[0] = (1.0 - _LAM) * ring[prv] + _LAM * headbuf[...]

    @pl.when(jnp.logical_and(g < _B, at_head == 1))
    def _():
        headbuf[...] = ring[slot]

    @pl.when(g == 0)
    def _():
        oy_ref[...] = jnp.dot(m_ref[...], y_ref[...], preferred_element_type=jnp.float32)


def kernel(x, y):
    grid_spec = pltpu.PrefetchScalarGridSpec(
        num_scalar_prefetch=3,
        grid=(_G,),
        in_specs=[
            pl.BlockSpec(memory_space=pl.ANY),
            pl.BlockSpec((_B, _B), lambda g, fsrc, head, dst: (0, 0)),
            pl.BlockSpec((_B, _NL), lambda g, fsrc, head, dst: (0, 0)),
        ],
        out_specs=[
            pl.BlockSpec((1, _C, _T), lambda g, fsrc, head, dst: (dst[g], 0, 0)),
            pl.BlockSpec((_B, _NL), lambda g, fsrc, head, dst: (0, 0)),
        ],
        scratch_shapes=[
            pltpu.VMEM((3, _C, _T), jnp.float32),
            pltpu.VMEM((_C, _T), jnp.float32),
            pltpu.SemaphoreType.DMA((3,)),
        ],
    )
    ox, oy = pl.pallas_call(
        _body,
        grid_spec=grid_spec,
        out_shape=[
            jax.ShapeDtypeStruct((_B, _C, _T), jnp.float32),
            jax.ShapeDtypeStruct((_B, _NL), jnp.float32),
        ],
    )(jnp.asarray(_FSRC), jnp.asarray(_HEAD), jnp.asarray(_DST), x, jnp.asarray(_MIX), y)
    return (ox, oy)


# speedup vs baseline: 1.5126x; 1.5126x over previous
import numpy as np
import jax
import jax.numpy as jnp
from jax.experimental import pallas as pl
from jax.experimental.pallas import tpu as pltpu

_B, _C, _T = 128, 128, 2048
_NL = 527


def _mix_plan():
    rs = np.random.RandomState(seed=1234)
    rs.uniform()
    rs.uniform()
    lam = rs.beta(0.3, 0.3)
    perm = rs.permutation(_B)
    order, is_head = [], []
    visited = np.zeros(_B, bool)
    for s in range(_B):
        if visited[s]:
            continue
        e = s
        first = True
        while not visited[e]:
            visited[e] = True
            order.append(int(e))
            is_head.append(1 if first else 0)
            first = False
            e = int(perm[e])
    fsrc = np.asarray(order + [0], np.int32)
    head = np.asarray(is_head + [1], np.int32)
    dst = np.asarray([order[0]] + order, np.int32)
    m = np.zeros((_B, _B), np.float32)
    m[np.arange(_B), np.arange(_B)] += np.float32(1.0 - lam)
    m[np.arange(_B), perm] += np.float32(lam)
    return float(lam), m, fsrc, head, dst


_LAM, _MIX, _FSRC, _HEAD, _DST = _mix_plan()
_G = len(_FSRC)


def _body(fsrc_ref, head_ref, dst_ref, x_hbm, m_ref, y_ref, ox_ref, oy_ref,
          ring, headbuf, sems):
    g = pl.program_id(0)
    slot = jax.lax.rem(g, 6)
    nxt = jax.lax.rem(g + 4, 6)
    prv = jax.lax.rem(g + 5, 6)
    nxt_src = fsrc_ref[jnp.minimum(g + 4, _G - 1)]
    at_head = head_ref[g]

    @pl.when(g == 0)
    def _():
        for k in range(5):
            pltpu.make_async_copy(x_hbm.at[fsrc_ref[k]], ring.at[k], sems.at[k]).start()

    @pl.when(jnp.logical_and(g >= 1, g + 4 < _B))
    def _():
        pltpu.make_async_copy(x_hbm.at[nxt_src], ring.at[nxt], sems.at[nxt]).start()

    @pl.when(g < _B)
    def _():
        pltpu.make_async_copy(x_hbm.at[fsrc_ref[g]], ring.at[slot], sems.at[slot]).wait()

    @pl.when(jnp.logical_and(g > 0, at_head == 0))
    def _():
        ox_ref[0] = (1.0 - _LAM) * ring[prv] + _LAM * ring[slot]

    @pl.when(jnp.logical_and(g > 0, at_head == 1))
    def _():
        ox_ref[0] = (1.0 - _LAM) * ring[prv] + _LAM * headbuf[...]

    @pl.when(jnp.logical_and(g < _B, at_head == 1))
    def _():
        headbuf[...] = ring[slot]

    @pl.when(g == 0)
    def _():
        oy_ref[...] = jnp.dot(m_ref[...], y_ref[...], preferred_element_type=jnp.float32)


def kernel(x, y):
    grid_spec = pltpu.PrefetchScalarGridSpec(
        num_scalar_prefetch=3,
        grid=(_G,),
        in_specs=[
            pl.BlockSpec(memory_space=pl.ANY),
            pl.BlockSpec((_B, _B), lambda g, fsrc, head, dst: (0, 0)),
            pl.BlockSpec((_B, _NL), lambda g, fsrc, head, dst: (0, 0)),
        ],
        out_specs=[
            pl.BlockSpec((1, _C, _T), lambda g, fsrc, head, dst: (dst[g], 0, 0)),
            pl.BlockSpec((_B, _NL), lambda g, fsrc, head, dst: (0, 0)),
        ],
        scratch_shapes=[
            pltpu.VMEM((6, _C, _T), jnp.float32),
            pltpu.VMEM((_C, _T), jnp.float32),
            pltpu.SemaphoreType.DMA((6,)),
        ],
    )
    ox, oy = pl.pallas_call(
        _body,
        grid_spec=grid_spec,
        out_shape=[
            jax.ShapeDtypeStruct((_B, _C, _T), jnp.float32),
            jax.ShapeDtypeStruct((_B, _NL), jnp.float32),
        ],
    )(jnp.asarray(_FSRC), jnp.asarray(_HEAD), jnp.asarray(_DST), x, jnp.asarray(_MIX), y)
    return (ox, oy)


# speedup vs baseline: 1.5159x; 1.0022x over previous
import numpy as np
import jax
import jax.numpy as jnp
from jax.experimental import pallas as pl
from jax.experimental.pallas import tpu as pltpu

_B, _C, _T = 128, 128, 2048
_NL = 527


def _mix_plan():
    rs = np.random.RandomState(seed=1234)
    rs.uniform()
    rs.uniform()
    lam = rs.beta(0.3, 0.3)
    perm = rs.permutation(_B)
    order, is_head = [], []
    visited = np.zeros(_B, bool)
    for s in range(_B):
        if visited[s]:
            continue
        e = s
        first = True
        while not visited[e]:
            visited[e] = True
            order.append(int(e))
            is_head.append(1 if first else 0)
            first = False
            e = int(perm[e])
    fsrc = np.asarray(order + [0], np.int32)
    head = np.asarray(is_head + [1], np.int32)
    dst = np.asarray([order[0]] + order, np.int32)
    m = np.zeros((_B, _B), np.float32)
    m[np.arange(_B), np.arange(_B)] += np.float32(1.0 - lam)
    m[np.arange(_B), perm] += np.float32(lam)
    return float(lam), m, fsrc, head, dst


_LAM, _MIX, _FSRC, _HEAD, _DST = _mix_plan()
_G = len(_FSRC)


def _body(fsrc_ref, head_ref, dst_ref, x_hbm, m_ref, y_ref, ox_ref, oy_ref,
          ring, headbuf, sems):
    g = pl.program_id(0)
    slot = jax.lax.rem(g, 10)
    nxt = jax.lax.rem(g + 8, 10)
    prv = jax.lax.rem(g + 9, 10)
    nxt_src = fsrc_ref[jnp.minimum(g + 8, _G - 1)]
    at_head = head_ref[g]

    @pl.when(g == 0)
    def _():
        for k in range(9):
            pltpu.make_async_copy(x_hbm.at[fsrc_ref[k]], ring.at[k], sems.at[k]).start()

    @pl.when(jnp.logical_and(g >= 1, g + 8 < _B))
    def _():
        pltpu.make_async_copy(x_hbm.at[nxt_src], ring.at[nxt], sems.at[nxt]).start()

    @pl.when(g < _B)
    def _():
        pltpu.make_async_copy(x_hbm.at[fsrc_ref[g]], ring.at[slot], sems.at[slot]).wait()

    @pl.when(jnp.logical_and(g > 0, at_head == 0))
    def _():
        ox_ref[0] = (1.0 - _LAM) * ring[prv] + _LAM * ring[slot]

    @pl.when(jnp.logical_and(g > 0, at_head == 1))
    def _():
        ox_ref[0] = (1.0 - _LAM) * ring[prv] + _LAM * headbuf[...]

    @pl.when(jnp.logical_and(g < _B, at_head == 1))
    def _():
        headbuf[...] = ring[slot]

    @pl.when(g == 0)
    def _():
        oy_ref[...] = jnp.dot(m_ref[...], y_ref[...], preferred_element_type=jnp.float32)


def kernel(x, y):
    grid_spec = pltpu.PrefetchScalarGridSpec(
        num_scalar_prefetch=3,
        grid=(_G,),
        in_specs=[
            pl.BlockSpec(memory_space=pl.ANY),
            pl.BlockSpec((_B, _B), lambda g, fsrc, head, dst: (0, 0)),
            pl.BlockSpec((_B, _NL), lambda g, fsrc, head, dst: (0, 0)),
        ],
        out_specs=[
            pl.BlockSpec((1, _C, _T), lambda g, fsrc, head, dst: (dst[g], 0, 0)),
            pl.BlockSpec((_B, _NL), lambda g, fsrc, head, dst: (0, 0)),
        ],
        scratch_shapes=[
            pltpu.VMEM((10, _C, _T), jnp.float32),
            pltpu.VMEM((_C, _T), jnp.float32),
            pltpu.SemaphoreType.DMA((10,)),
        ],
    )
    ox, oy = pl.pallas_call(
        _body,
        grid_spec=grid_spec,
        out_shape=[
            jax.ShapeDtypeStruct((_B, _C, _T), jnp.float32),
            jax.ShapeDtypeStruct((_B, _NL), jnp.float32),
        ],
    )(jnp.asarray(_FSRC), jnp.asarray(_HEAD), jnp.asarray(_DST), x, jnp.asarray(_MIX), y)
    return (ox, oy)


# speedup vs baseline: 1.5602x; 1.0293x over previous
import numpy as np
import jax
import jax.numpy as jnp
from jax.experimental import pallas as pl
from jax.experimental.pallas import tpu as pltpu

_B, _C, _T = 128, 128, 2048
_NL = 527


def _mix_plan():
    rs = np.random.RandomState(seed=1234)
    rs.uniform()
    rs.uniform()
    lam = rs.beta(0.3, 0.3)
    perm = rs.permutation(_B)
    order, is_head = [], []
    visited = np.zeros(_B, bool)
    for s in range(_B):
        if visited[s]:
            continue
        e = s
        first = True
        while not visited[e]:
            visited[e] = True
            order.append(int(e))
            is_head.append(1 if first else 0)
            first = False
            e = int(perm[e])
    pad = 7
    fsrc = np.asarray(order + [0] * pad, np.int32)
    head = np.asarray(is_head + [1] + [0] * (pad - 1), np.int32)
    dst = np.asarray([order[0]] + order + [0] * (pad - 1), np.int32)
    m = np.zeros((_B, _B), np.float32)
    m[np.arange(_B), np.arange(_B)] += np.float32(1.0 - lam)
    m[np.arange(_B), perm] += np.float32(lam)
    return float(lam), m, fsrc, head, dst


_LAM, _MIX, _FSRC, _HEAD, _DST = _mix_plan()
_G = _B + 7
_NIN = 10
_NOUT = 6


def _body(fsrc_ref, head_ref, dst_ref, x_hbm, m_ref, y_ref, ox_hbm, oy_ref,
          ring, stage, headbuf, isems, osems):
    g = pl.program_id(0)
    slot = jax.lax.rem(g, _NIN)
    nxt = jax.lax.rem(g + 8, _NIN)
    prv = jax.lax.rem(g + _NIN - 1, _NIN)
    oslot = jax.lax.rem(g, _NOUT)
    nxt_src = fsrc_ref[jnp.minimum(g + 8, _G - 1)]
    at_head = head_ref[g]

    @pl.when(g == 0)
    def _():
        for k in range(9):
            pltpu.make_async_copy(x_hbm.at[fsrc_ref[k]], ring.at[k], isems.at[k]).start()

    @pl.when(jnp.logical_and(g >= 1, g + 8 < _B))
    def _():
        pltpu.make_async_copy(x_hbm.at[nxt_src], ring.at[nxt], isems.at[nxt]).start()

    @pl.when(jnp.logical_and(g >= _NOUT + 1, g - _NOUT <= _B))
    def _():
        pltpu.make_async_copy(
            stage.at[oslot], ox_hbm.at[dst_ref[g - _NOUT]], osems.at[oslot]
        ).wait()

    @pl.when(g < _B)
    def _():
        pltpu.make_async_copy(x_hbm.at[fsrc_ref[g]], ring.at[slot], isems.at[slot]).wait()

    @pl.when(jnp.logical_and(jnp.logical_and(g > 0, g <= _B), at_head == 0))
    def _():
        stage[oslot] = (1.0 - _LAM) * ring[prv] + _LAM * ring[slot]
        pltpu.make_async_copy(stage.at[oslot], ox_hbm.at[dst_ref[g]], osems.at[oslot]).start()

    @pl.when(jnp.logical_and(jnp.logical_and(g > 0, g <= _B), at_head == 1))
    def _():
        stage[oslot] = (1.0 - _LAM) * ring[prv] + _LAM * headbuf[...]
        pltpu.make_async_copy(stage.at[oslot], ox_hbm.at[dst_ref[g]], osems.at[oslot]).start()

    @pl.when(jnp.logical_and(g < _B, at_head == 1))
    def _():
        headbuf[...] = ring[slot]

    @pl.when(g == 0)
    def _():
        oy_ref[...] = jnp.dot(m_ref[...], y_ref[...], preferred_element_type=jnp.float32)


def kernel(x, y):
    grid_spec = pltpu.PrefetchScalarGridSpec(
        num_scalar_prefetch=3,
        grid=(_G,),
        in_specs=[
            pl.BlockSpec(memory_space=pl.ANY),
            pl.BlockSpec((_B, _B), lambda g, fsrc, head, dst: (0, 0)),
            pl.BlockSpec((_B, _NL), lambda g, fsrc, head, dst: (0, 0)),
        ],
        out_specs=[
            pl.BlockSpec(memory_space=pl.ANY),
            pl.BlockSpec((_B, _NL), lambda g, fsrc, head, dst: (0, 0)),
        ],
        scratch_shapes=[
            pltpu.VMEM((_NIN, _C, _T), jnp.float32),
            pltpu.VMEM((_NOUT, _C, _T), jnp.float32),
            pltpu.VMEM((_C, _T), jnp.float32),
            pltpu.SemaphoreType.DMA((_NIN,)),
            pltpu.SemaphoreType.DMA((_NOUT,)),
        ],
    )
    ox, oy = pl.pallas_call(
        _body,
        grid_spec=grid_spec,
        out_shape=[
            jax.ShapeDtypeStruct((_B, _C, _T), jnp.float32),
            jax.ShapeDtypeStruct((_B, _NL), jnp.float32),
        ],
    )(jnp.asarray(_FSRC), jnp.asarray(_HEAD), jnp.asarray(_DST), x, jnp.asarray(_MIX), y)
    return (ox, oy)
